# baseline (speedup 1.0000x reference)
import functools

import jax
import jax.numpy as jnp
from jax import lax
from jax.experimental import pallas as pl
from jax.experimental.pallas import tpu as pltpu
from jax.experimental.pallas import tpu_sc as plsc

_H = 32
_W = 32
_HW = _H * _W
_DH = 384
_D = 768
_L = 16
_NC = 2
_NS = 16
_WINDOW = 8


def _sc_body(batch, row_hbm, col_hbm, idx_hbm, idxpad_hbm, out_hbm,
             idx_v, idxpad_v, colrows_v, rowone_v, buf_v, gsem, osem):
    wid = lax.axis_index("s") * _NC + lax.axis_index("c")
    pltpu.sync_copy(idx_hbm, idx_v)
    pltpu.sync_copy(idxpad_hbm, idxpad_v)
    pltpu.async_copy(col_hbm.at[idx_v], colrows_v, gsem).wait()
    pltpu.async_copy(row_hbm.at[idxpad_v.at[pl.ds(8 * wid, 1)]], rowone_v, gsem).wait()

    row_regs = [rowone_v[0, pl.ds(_L * k, _L)] for k in range(_DH // _L)]
    for w in range(_W):
        for k in range(_DH // _L):
            buf_v[w, pl.ds(_L * k, _L)] = colrows_v[w, pl.ds(_L * k, _L)]
        for k in range(_DH // _L):
            buf_v[w, pl.ds(_DH + _L * k, _L)] = row_regs[k]

    copies = [
        pltpu.make_async_copy(buf_v, out_hbm.at[b, pl.ds(_H * wid, _H), :], osem)
        for b in range(batch)
    ]
    for b in range(batch):
        copies[b].start()
        if b >= _WINDOW:
            copies[b - _WINDOW].wait()
    for b in range(max(batch - _WINDOW, 0), batch):
        copies[b].wait()


def kernel(x, height, width, row_embed, col_embed):
    batch = x.shape[0]
    zero = (jnp.asarray(height, jnp.int32) - _H) + (jnp.asarray(width, jnp.int32) - _W)
    idx = jnp.arange(_W, dtype=jnp.int32) + zero
    idxpad = jnp.repeat(idx, 8)
    mesh = plsc.VectorSubcoreMesh(core_axis_name="c", subcore_axis_name="s")
    k = functools.partial(
        pl.kernel,
        mesh=mesh,
        out_type=jax.ShapeDtypeStruct((batch, _HW, _D), jnp.float32),
        scratch_types=[
            pltpu.VMEM((_W,), jnp.int32),
            pltpu.VMEM((_W * 8,), jnp.int32),
            pltpu.VMEM((_W, _DH), jnp.float32),
            pltpu.VMEM((1, _DH), jnp.float32),
            pltpu.VMEM((_W, _D), jnp.float32),
            pltpu.SemaphoreType.DMA,
            pltpu.SemaphoreType.DMA,
        ],
    )(functools.partial(_sc_body, batch))
    return k(row_embed, col_embed, idx, idxpad)
